# baseline (speedup 1.0000x reference)
import jax
import jax.numpy as jnp
from jax.experimental import pallas as pl
from jax.experimental.pallas import tpu as pltpu
from jax.experimental.pallas import tpu_sc as plsc

_GATHER_WINDOW = 128
_ROW_SPLIT = 2
_BATCH_CHUNKS = 1


def _sc_gather_rows(table, idx_flat, width):
    n = idx_flat.shape[0]
    indices = idx_flat.reshape(1, n)
    mesh = plsc.VectorSubcoreMesh(core_axis_name="core", subcore_axis_name="subcore")

    @pl.kernel(
        out_type=jax.ShapeDtypeStruct((n, width), table.dtype),
        mesh=mesh,
    )
    def gather_kernel(x_hbm, i_hbm, o_hbm):
        def body(i_vmem, o_vmem):
            pltpu.sync_copy(x_hbm.at[i_vmem.at[0]], o_vmem)

        pltpu.emit_pipeline(
            body,
            grid=(n // _GATHER_WINDOW,),
            in_specs=[pl.BlockSpec((1, _GATHER_WINDOW), lambda i: (0, i))],
            out_specs=[pl.BlockSpec((_GATHER_WINDOW, width), lambda i: (i, 0))],
            core_axis_name=("core", "subcore"),
            dimension_semantics=(pltpu.PARALLEL,),
        )(i_hbm, o_hbm)

    return gather_kernel(table, indices)


def _rhs_t_dot(a, w):
    return jax.lax.dot_general(a, w, (((1,), (1,)), ((), ())),
                               preferred_element_type=jnp.float32)


def _lstm_decode(features, embeds_rest, Wih, Whh, b, Wout, b_out, h0, c0):
    Lm1, B, E = embeds_rest.shape
    L = Lm1 + 1
    H = Whh.shape[1]
    V = Wout.shape[0]

    def step_kernel(x_ref, feat_ref, wih_ref, whh_ref, b_ref, wout_ref,
                    bout_ref, h0_ref, c0_ref, out_ref, h_ref, c_ref):
        t = pl.program_id(0)

        @pl.when(t == 0)
        def _init():
            h_ref[...] = h0_ref[...]
            c_ref[...] = c0_ref[...]

        R = B // _BATCH_CHUNKS
        for r in range(_BATCH_CHUNKS):
            rows = pl.ds(r * R, R)
            h_prev = h_ref[rows, :]

            @pl.when(t > 0)
            def _project_prev(h_prev=h_prev, rows=rows):
                out_ref[0, rows, :] = (
                    _rhs_t_dot(h_prev, wout_ref[...]) + bout_ref[...]
                )

            @pl.when(t < L)
            def _recurrence(h_prev=h_prev, rows=rows):
                x = jnp.where(t == 0, feat_ref[rows, :],
                              x_ref[0, rows, :])
                gates = (
                    _rhs_t_dot(x, wih_ref[...])
                    + _rhs_t_dot(h_prev, whh_ref[...])
                    + b_ref[...]
                )
                i = jax.nn.sigmoid(gates[:, 0 * H:1 * H])
                f = jax.nn.sigmoid(gates[:, 1 * H:2 * H])
                g = jnp.tanh(gates[:, 2 * H:3 * H])
                o = jax.nn.sigmoid(gates[:, 3 * H:4 * H])
                c = f * c_ref[rows, :] + i * g
                h = o * jnp.tanh(c)
                c_ref[rows, :] = c
                h_ref[rows, :] = h

    return pl.pallas_call(
        step_kernel,
        grid=(L + 1,),
        in_specs=[
            pl.BlockSpec((1, B, E),
                         lambda t: (jnp.clip(t - 1, 0, Lm1 - 1), 0, 0)),
            pl.BlockSpec((B, E), lambda t: (0, 0)),
            pl.BlockSpec((4 * H, E), lambda t: (0, 0)),
            pl.BlockSpec((4 * H, H), lambda t: (0, 0)),
            pl.BlockSpec((1, 4 * H), lambda t: (0, 0)),
            pl.BlockSpec((V, H), lambda t: (0, 0)),
            pl.BlockSpec((1, V), lambda t: (0, 0)),
            pl.BlockSpec((B, H), lambda t: (0, 0)),
            pl.BlockSpec((B, H), lambda t: (0, 0)),
        ],
        out_specs=pl.BlockSpec((1, B, V),
                               lambda t: (jnp.maximum(t, 1) - 1, 0, 0)),
        out_shape=jax.ShapeDtypeStruct((L, B, V), jnp.float32),
        scratch_shapes=[
            pltpu.VMEM((B, H), jnp.float32),
            pltpu.VMEM((B, H), jnp.float32),
        ],
        compiler_params=pltpu.CompilerParams(
            dimension_semantics=("arbitrary",),
        ),
    )(embeds_rest, features, Wih, Whh, b, Wout, b_out, h0, c0)


def _decode_batch(features, captions, emb, W_ih, W_hh, b, W_out, b_out2,
                  h0_2d, c0_2d):
    B, L = captions.shape
    E = emb.shape[1]

    idx = jnp.transpose(captions[:, :-1]).reshape(-1)
    s = _ROW_SPLIT
    idx_split = (idx[:, None] * s
                 + jnp.arange(s, dtype=idx.dtype)[None, :]).reshape(-1)
    table = emb.reshape(emb.shape[0] * s, E // s)
    gathered = _sc_gather_rows(table, idx_split, E // s)
    embeds_rest = gathered.reshape(L - 1, B, E)

    logits_tm = _lstm_decode(features, embeds_rest, W_ih, W_hh, b,
                             W_out, b_out2, h0_2d, c0_2d)
    return jnp.transpose(logits_tm, (1, 0, 2))


def kernel(features, captions, emb, W_ih, W_hh, b_ih, b_hh, W_out, b_out, h0, c0):
    b = (b_ih + b_hh).reshape(1, -1)
    b_out2 = b_out.reshape(1, -1)
    args = (features, captions, emb, W_ih, W_hh, b, W_out, b_out2,
            h0[0], c0[0])

    devs = jax.devices()
    if len(devs) >= 2 and captions.shape[0] % 2 == 0:
        P = jax.sharding.PartitionSpec
        mesh = jax.sharding.Mesh(devs[:2], ("d",))
        rep = P(*([None] * 2))
        return jax.shard_map(
            _decode_batch,
            mesh=mesh,
            in_specs=(P("d", None), P("d", None), rep, rep, rep, rep, rep,
                      rep, P("d", None), P("d", None)),
            out_specs=P("d", None, None),
            check_vma=False,
        )(*args)
    return _decode_batch(*args)


# speedup vs baseline: 2.9043x; 2.9043x over previous
import jax
import jax.numpy as jnp
from jax.experimental import pallas as pl
from jax.experimental.pallas import tpu as pltpu
from jax.experimental.pallas import tpu_sc as plsc

_GATHER_WINDOW = 128
_ROW_SPLIT = 2
_BATCH_CHUNKS = 1


def _sc_gather_rows(table, idx_flat, width):
    n = idx_flat.shape[0]
    indices = idx_flat.reshape(1, n)
    mesh = plsc.VectorSubcoreMesh(core_axis_name="core", subcore_axis_name="subcore")

    @pl.kernel(
        out_type=jax.ShapeDtypeStruct((n, width), table.dtype),
        mesh=mesh,
    )
    def gather_kernel(x_hbm, i_hbm, o_hbm):
        def body(i_vmem, o_vmem):
            pltpu.sync_copy(x_hbm.at[i_vmem.at[0]], o_vmem)

        pltpu.emit_pipeline(
            body,
            grid=(n // _GATHER_WINDOW,),
            in_specs=[pl.BlockSpec((1, _GATHER_WINDOW), lambda i: (0, i))],
            out_specs=[pl.BlockSpec((_GATHER_WINDOW, width), lambda i: (i, 0))],
            core_axis_name=("core", "subcore"),
            dimension_semantics=(pltpu.PARALLEL,),
        )(i_hbm, o_hbm)

    return gather_kernel(table, indices)


def _rhs_t_dot(a, w):
    return jax.lax.dot_general(a, w, (((1,), (1,)), ((), ())),
                               preferred_element_type=jnp.float32)


def _lstm_decode(features, embeds_rest, Wih, Whh, b, Wout, b_out, h0, c0):
    Lm1, B, E = embeds_rest.shape
    L = Lm1 + 1
    H = Whh.shape[1]
    V = Wout.shape[0]

    def step_kernel(x_ref, feat_ref, wih_ref, whh_ref, b_ref, wout_ref,
                    bout_ref, h0_ref, c0_ref, out_ref, h_ref, c_ref):
        t = pl.program_id(0)

        @pl.when(t == 0)
        def _init():
            h_ref[...] = h0_ref[...]
            c_ref[...] = c0_ref[...]

        R = B // _BATCH_CHUNKS
        for r in range(_BATCH_CHUNKS):
            rows = pl.ds(r * R, R)
            h_prev = h_ref[rows, :]

            out_ref[0, rows, :] = (
                _rhs_t_dot(h_prev, wout_ref[...]) + bout_ref[...]
            )

            x = jnp.where(t == 0, feat_ref[rows, :], x_ref[0, rows, :])
            gates = (
                _rhs_t_dot(x, wih_ref[...])
                + _rhs_t_dot(h_prev, whh_ref[...])
                + b_ref[...]
            )
            i = jax.nn.sigmoid(gates[:, 0 * H:1 * H])
            f = jax.nn.sigmoid(gates[:, 1 * H:2 * H])
            g = jnp.tanh(gates[:, 2 * H:3 * H])
            o = jax.nn.sigmoid(gates[:, 3 * H:4 * H])
            c = f * c_ref[rows, :] + i * g
            h = o * jnp.tanh(c)
            c_ref[rows, :] = c
            h_ref[rows, :] = h

    return pl.pallas_call(
        step_kernel,
        grid=(L + 1,),
        in_specs=[
            pl.BlockSpec((1, B, E),
                         lambda t: (jnp.clip(t - 1, 0, Lm1 - 1), 0, 0)),
            pl.BlockSpec((B, E), lambda t: (0, 0)),
            pl.BlockSpec((4 * H, E), lambda t: (0, 0)),
            pl.BlockSpec((4 * H, H), lambda t: (0, 0)),
            pl.BlockSpec((1, 4 * H), lambda t: (0, 0)),
            pl.BlockSpec((V, H), lambda t: (0, 0)),
            pl.BlockSpec((1, V), lambda t: (0, 0)),
            pl.BlockSpec((B, H), lambda t: (0, 0)),
            pl.BlockSpec((B, H), lambda t: (0, 0)),
        ],
        out_specs=pl.BlockSpec((1, B, V),
                               lambda t: (jnp.maximum(t, 1) - 1, 0, 0)),
        out_shape=jax.ShapeDtypeStruct((L, B, V), jnp.float32),
        scratch_shapes=[
            pltpu.VMEM((B, H), jnp.float32),
            pltpu.VMEM((B, H), jnp.float32),
        ],
        compiler_params=pltpu.CompilerParams(
            dimension_semantics=("arbitrary",),
        ),
    )(embeds_rest, features, Wih, Whh, b, Wout, b_out, h0, c0)


def _decode_batch(features, captions, emb, W_ih, W_hh, b, W_out, b_out2,
                  h0_2d, c0_2d):
    B, L = captions.shape
    E = emb.shape[1]

    idx = jnp.transpose(captions[:, :-1]).reshape(-1)
    s = _ROW_SPLIT
    idx_split = (idx[:, None] * s
                 + jnp.arange(s, dtype=idx.dtype)[None, :]).reshape(-1)
    table = emb.reshape(emb.shape[0] * s, E // s)
    gathered = _sc_gather_rows(table, idx_split, E // s)
    embeds_rest = gathered.reshape(L - 1, B, E)

    logits_tm = _lstm_decode(features, embeds_rest, W_ih, W_hh, b,
                             W_out, b_out2, h0_2d, c0_2d)
    return jnp.transpose(logits_tm, (1, 0, 2))


def kernel(features, captions, emb, W_ih, W_hh, b_ih, b_hh, W_out, b_out, h0, c0):
    b = (b_ih + b_hh).reshape(1, -1)
    b_out2 = b_out.reshape(1, -1)
    args = (features, captions, emb, W_ih, W_hh, b, W_out, b_out2,
            h0[0], c0[0])

    return _decode_batch(*args)
